# Initial kernel scaffold; baseline (speedup 1.0000x reference)
import jax
import jax.numpy as jnp
from jax.experimental import pallas as pl


def kernel(input_ids, token_type_ids, word_table, pos_table, tok_table, ln_gamma, ln_beta, W_lin, b_lin):
    raise NotImplementedError("write your pallas kernel here")


# speedup vs baseline: 1.9292x; 1.9292x over previous
import functools

import jax
import jax.numpy as jnp
from jax import lax
from jax.experimental import pallas as pl
from jax.experimental.pallas import tpu as pltpu
from jax.experimental.pallas import tpu_sc as plsc

SEQ = 8192
D = 768
EPS = 1e-12

NC = 2
NS = 16
NW = NC * NS
RPW = SEQ // NW
CHUNK = 64
NCH = RPW // CHUNK

@functools.lru_cache(maxsize=None)
def _make_sc_gather():
    mesh = plsc.VectorSubcoreMesh(core_axis_name="c", subcore_axis_name="s")
    return functools.partial(
        pl.kernel,
        out_type=jax.ShapeDtypeStruct((SEQ, D), jnp.float32),
        mesh=mesh,
        scratch_types=[
            pltpu.VMEM((NCH, CHUNK), jnp.int32),
            pltpu.VMEM((2, CHUNK, D), jnp.float32),
            pltpu.SemaphoreType.DMA,
            pltpu.SemaphoreType.DMA,
        ],
    )(_sc_gather_body)


def _sc_gather_body(ids_hbm, table_hbm, out_hbm, idx_v, rows_v, sem0, sem1):
    wid = lax.axis_index("s") * NC + lax.axis_index("c")
    base = wid * RPW
    pltpu.sync_copy(ids_hbm.at[wid], idx_v)
    sems = (sem0, sem1)
    copies = [
        pltpu.async_copy(table_hbm.at[idx_v.at[0]], rows_v.at[0], sems[0])
    ]
    for c in range(NCH):
        if c + 1 < NCH:
            copies.append(
                pltpu.async_copy(
                    table_hbm.at[idx_v.at[c + 1]],
                    rows_v.at[(c + 1) % 2],
                    sems[(c + 1) % 2],
                )
            )
        copies[c].wait()
        pltpu.sync_copy(
            rows_v.at[c % 2], out_hbm.at[pl.ds(base + c * CHUNK, CHUNK)]
        )


BT = 512


def _tc_body(word_ref, pos_ref, tt_ref, tok_ref, gam_ref, bet_ref, w_ref,
             b_ref, out_ref):
    x = word_ref[...] + pos_ref[...]
    t = tt_ref[...].astype(jnp.float32)
    tok0 = tok_ref[0:1, :]
    tok1 = tok_ref[1:2, :]
    x = x + tok0 + t * (tok1 - tok0)
    mean = jnp.mean(x, axis=-1, keepdims=True)
    xc = x - mean
    var = jnp.mean(xc * xc, axis=-1, keepdims=True)
    y = xc * lax.rsqrt(var + EPS) * gam_ref[...] + bet_ref[...]
    acc = lax.dot_general(
        y.astype(jnp.bfloat16), w_ref[...],
        dimension_numbers=(((1,), (1,)), ((), ())),
        preferred_element_type=jnp.float32,
    )
    out_ref[...] = acc + b_ref[...]


_tc_call = pl.pallas_call(
    _tc_body,
    grid=(SEQ // BT,),
    in_specs=[
        pl.BlockSpec((BT, D), lambda i: (i, 0)),
        pl.BlockSpec((BT, D), lambda i: (i, 0)),
        pl.BlockSpec((BT, 1), lambda i: (i, 0)),
        pl.BlockSpec((2, D), lambda i: (0, 0)),
        pl.BlockSpec((1, D), lambda i: (0, 0)),
        pl.BlockSpec((1, D), lambda i: (0, 0)),
        pl.BlockSpec((D, D), lambda i: (0, 0)),
        pl.BlockSpec((1, D), lambda i: (0, 0)),
    ],
    out_specs=pl.BlockSpec((BT, D), lambda i: (i, 0)),
    out_shape=jax.ShapeDtypeStruct((SEQ, D), jnp.float32),
)


def kernel(input_ids, token_type_ids, word_table, pos_table, tok_table,
           ln_gamma, ln_beta, W_lin, b_lin):
    ids3 = input_ids.reshape(NW, NCH, CHUNK)
    word_emb = _make_sc_gather()(ids3, word_table)
    out = _tc_call(
        word_emb,
        pos_table[:SEQ],
        token_type_ids.reshape(SEQ, 1),
        tok_table,
        ln_gamma.reshape(1, D),
        ln_beta.reshape(1, D),
        W_lin.astype(jnp.bfloat16),
        b_lin.reshape(1, D),
    )
    return out.reshape(1, SEQ, D)
